# baseline (speedup 1.0000x reference)
import functools

import jax
import jax.numpy as jnp
from jax import lax
from jax.experimental import pallas as pl
from jax.experimental.pallas import tpu as pltpu
from jax.experimental.pallas import tpu_sc as plsc

_K = 32
_N_ITER = 13
_N_NEWTON = 2
_N_TAIL = 4
_CLIP = 0.02
_C_LO0 = 256.0
_SEED_T = 2.728
_ROW_BLOCK = 200

_NW = 32
_RPW = 320


def _sc_bounds(n, m):
    nvec = m // 16
    half = (nvec // 2) * 16

    @functools.partial(
        pl.kernel,
        mesh=plsc.VectorSubcoreMesh(core_axis_name="c", subcore_axis_name="s"),
        out_type=(
            jax.ShapeDtypeStruct((_NW * _RPW,), jnp.float32),
            jax.ShapeDtypeStruct((_NW * _RPW,), jnp.float32),
        ),
        scratch_types=[
            pltpu.VMEM((m,), jnp.float32),
            pltpu.VMEM((_RPW,), jnp.float32),
            pltpu.VMEM((_RPW,), jnp.float32),
        ],
    )
    def sc_kernel(adj_hbm, lo_hbm, hi_hbm, rowbuf, res_lo, res_hi, *rest):
        wid = lax.axis_index("s") * 2 + lax.axis_index("c")
        base = wid * _RPW
        lanes = lax.iota(jnp.int32, 16)

        def lane_perm(v, perm):
            return lax.gather(
                v,
                perm.reshape(16, 1),
                lax.GatherDimensionNumbers(
                    offset_dims=(),
                    collapsed_slice_dims=(0,),
                    start_index_map=(0,),
                ),
                slice_sizes=(1,),
                mode=lax.GatherScatterMode.PROMISE_IN_BOUNDS,
            )

        def hreduce(v, op):
            for s in (8, 4, 2, 1):
                v = op(v, lane_perm(v, jnp.bitwise_xor(lanes, s)))
            return v

        def row_body(i, carry):
            acc_lo, acc_hi = carry
            r = jnp.minimum(base + i, n - 1)
            pltpu.sync_copy(adj_hbm.at[r], rowbuf)

            def vmax_half(lo_ix, cnt):
                def step(j, acc):
                    return jnp.maximum(acc, rowbuf[pl.ds(lo_ix + j * 16, 16)])
                init = rowbuf[pl.ds(lo_ix, 16)]
                return lax.fori_loop(1, cnt, step, init)

            ga = vmax_half(0, nvec // 2)
            gb = vmax_half(half, nvec - nvec // 2)
            t_lo = hreduce(jnp.minimum(ga, gb), jnp.minimum)
            t_hi = hreduce(jnp.maximum(ga, gb), jnp.maximum)

            lane = i % 16
            acc_lo = jnp.where(lanes == lane, t_lo, acc_lo)
            acc_hi = jnp.where(lanes == lane, t_hi, acc_hi)

            @pl.when(lane == 15)
            def _():
                res_lo[pl.ds(i - 15, 16)] = acc_lo
                res_hi[pl.ds(i - 15, 16)] = acc_hi

            return acc_lo, acc_hi

        z = jnp.zeros((16,), jnp.float32)
        lax.fori_loop(0, _RPW, row_body, (z, z))
        pltpu.sync_copy(res_lo, lo_hbm.at[pl.ds(base, _RPW)])
        pltpu.sync_copy(res_hi, hi_hbm.at[pl.ds(base, _RPW)])

    return sc_kernel


def _tc_body(x_ref, lo_ref, hi_ref, o_ref):
    x = x_ref[...]
    r, m = x.shape
    lo = lo_ref[...]
    hi = hi_ref[...]

    c_lo = jnp.full((r, 1), _C_LO0, dtype=x.dtype)
    c_hi = jnp.ones((r, 1), dtype=x.dtype)
    l_tgt = jnp.log(jnp.float32(_K))
    t_prev = l_prev = None

    for it in range(_N_ITER):
        w = hi - lo
        if it == 0:
            mid = jnp.full((r, 1), jnp.float32(_SEED_T))
        elif it <= _N_NEWTON:
            mid = t_prev + (l_prev - l_tgt) / jnp.maximum(t_prev, 1.0)
        elif it < _N_ITER - _N_TAIL:
            l_lo = jnp.log(jnp.maximum(c_lo, 1.0))
            l_hi = jnp.log(jnp.maximum(c_hi, 0.25))
            denom = jnp.maximum(l_lo - l_hi, 1e-9)
            mid = lo + w * ((l_lo - l_tgt) / denom)
        else:
            mid = lo + 0.5 * w
        if it < _N_ITER - _N_TAIL:
            mid = jnp.clip(mid, lo + _CLIP * w, hi - _CLIP * w)
        cnt = jnp.sum(jnp.where(x >= mid, 1.0, 0.0), axis=1, keepdims=True)
        t_prev, l_prev = mid, jnp.log(jnp.maximum(cnt, 0.5))
        ge = cnt >= _K
        lo = jnp.where(ge, mid, lo)
        c_lo = jnp.where(ge, cnt, c_lo)
        hi = jnp.where(ge, hi, mid)
        c_hi = jnp.where(ge, c_hi, cnt)

    o_ref[...] = jnp.where(x >= lo, x, 0.0)


def kernel(adj):
    n, m = adj.shape
    lo, hi = _sc_bounds(n, m)(adj)
    lo = lo[:n].reshape(n, 1)
    hi = hi[:n].reshape(n, 1)
    grid = (n // _ROW_BLOCK,)
    return pl.pallas_call(
        _tc_body,
        grid=grid,
        in_specs=[
            pl.BlockSpec((_ROW_BLOCK, m), lambda i: (i, 0)),
            pl.BlockSpec((_ROW_BLOCK, 1), lambda i: (i, 0)),
            pl.BlockSpec((_ROW_BLOCK, 1), lambda i: (i, 0)),
        ],
        out_specs=pl.BlockSpec((_ROW_BLOCK, m), lambda i: (i, 0)),
        out_shape=jax.ShapeDtypeStruct((n, m), adj.dtype),
    )(adj, lo, hi)


# speedup vs baseline: 2.8750x; 2.8750x over previous
import jax
import jax.numpy as jnp
from jax.experimental import pallas as pl
from jax.experimental.pallas import tpu as pltpu

_K = 32
_N_ITER = 13
_N_NEWTON = 2
_N_TAIL = 4
_CLIP = 0.02
_C_LO0 = 256.0
_SEED_T = 2.728
_ROW_BLOCK = 400


def _body(x_ref, o_ref):
    x = x_ref[...]
    r, m = x.shape
    ngrp = m // 128

    gm = x[:, 0:128]
    for g in range(1, ngrp):
        gm = jnp.maximum(gm, x[:, g * 128:(g + 1) * 128])
    lo = jnp.min(gm, axis=1, keepdims=True)
    hi = jnp.max(gm, axis=1, keepdims=True)
    if m % 128:
        hi = jnp.maximum(hi, jnp.max(x[:, ngrp * 128:], axis=1, keepdims=True))

    c_lo = jnp.full((r, 1), _C_LO0, dtype=x.dtype)
    c_hi = jnp.ones((r, 1), dtype=x.dtype)
    l_tgt = jnp.log(jnp.float32(_K))
    t_prev = l_prev = None

    for it in range(_N_ITER):
        w = hi - lo
        if it == 0:
            mid = jnp.full((r, 1), jnp.float32(_SEED_T))
        elif it <= _N_NEWTON:
            mid = t_prev + (l_prev - l_tgt) / jnp.maximum(t_prev, 1.0)
        elif it < _N_ITER - _N_TAIL:
            l_lo = jnp.log(jnp.maximum(c_lo, 1.0))
            l_hi = jnp.log(jnp.maximum(c_hi, 0.25))
            denom = jnp.maximum(l_lo - l_hi, 1e-9)
            mid = lo + w * ((l_lo - l_tgt) / denom)
        else:
            mid = lo + 0.5 * w
        if it < _N_ITER - _N_TAIL:
            mid = jnp.clip(mid, lo + _CLIP * w, hi - _CLIP * w)
        cnt = jnp.sum(jnp.where(x >= mid, 1.0, 0.0), axis=1, keepdims=True)
        t_prev, l_prev = mid, jnp.log(jnp.maximum(cnt, 0.5))
        ge = cnt >= _K
        lo = jnp.where(ge, mid, lo)
        c_lo = jnp.where(ge, cnt, c_lo)
        hi = jnp.where(ge, hi, mid)
        c_hi = jnp.where(ge, c_hi, cnt)

    o_ref[...] = jnp.where(x >= lo, x, 0.0)


def kernel(adj):
    n, m = adj.shape
    grid = (n // _ROW_BLOCK,)
    return pl.pallas_call(
        _body,
        grid=grid,
        in_specs=[pl.BlockSpec((_ROW_BLOCK, m), lambda i: (i, 0))],
        out_specs=pl.BlockSpec((_ROW_BLOCK, m), lambda i: (i, 0)),
        out_shape=jax.ShapeDtypeStruct((n, m), adj.dtype),
        compiler_params=pltpu.CompilerParams(
            vmem_limit_bytes=128 * 1024 * 1024,
        ),
    )(adj)
